# Initial kernel scaffold; baseline (speedup 1.0000x reference)
import jax
import jax.numpy as jnp
from jax.experimental import pallas as pl


def kernel(trigger, action, target, operand_id, child_trigger, child_action, child_target, child_parent, T_trigger, T_effect, T_target, T_operand, W_leaf, b_leaf, W_n1, b_n1, W_n2, b_n2):
    raise NotImplementedError("write your pallas kernel here")


# speedup vs baseline: 2.3885x; 2.3885x over previous
import functools

import jax
import jax.numpy as jnp
from jax import lax
from jax.experimental import pallas as pl
from jax.experimental.pallas import tpu as pltpu
from jax.experimental.pallas import tpu_sc as plsc

H = 96
NP = 65536
NC = 262144
NCOMBO = 630
CPAD = 640
NW = 32
PB = 512
NBLK = NP // PB
PCH = 512
CH = 1024
LANES = 16


def _lut_kernel(combo_ref, wl_ref, bl_ref, wn1_ref, to_ref, bn1_ref,
                flut_ref, plut_ref, olut_ref):
    combo = combo_ref[...]
    leaf = jnp.maximum(
        jnp.dot(combo, wl_ref[...], preferred_element_type=jnp.float32)
        + bl_ref[...], 0.0)
    wb = wn1_ref[H:, :]
    wa = wn1_ref[:H, :]
    flut_ref[...] = jnp.dot(leaf, wb, preferred_element_type=jnp.float32)
    plut_ref[...] = jnp.dot(combo, wb, preferred_element_type=jnp.float32)
    olut_ref[...] = (
        jnp.dot(to_ref[...], wa, preferred_element_type=jnp.float32)
        + bn1_ref[...])


def _build_luts(combo_pad, w_leaf, b_leaf, w_n1, to_pad, b_n1):
    return pl.pallas_call(
        _lut_kernel,
        out_shape=(
            jax.ShapeDtypeStruct((CPAD, H), jnp.float32),
            jax.ShapeDtypeStruct((CPAD, H), jnp.float32),
            jax.ShapeDtypeStruct((16, H), jnp.float32),
        ),
    )(combo_pad, w_leaf, b_leaf.reshape(1, H), w_n1, to_pad,
      b_n1.reshape(1, H))


def _mlp_kernel(pre_ref, w2_ref, b2_ref, out_ref):
    h = jnp.maximum(pre_ref[...], 0.0)
    out_ref[...] = (
        jnp.dot(h, w2_ref[...], preferred_element_type=jnp.float32)
        + b2_ref[...])


def _out_mlp(pre, w2, b2):
    blk = 4096
    return pl.pallas_call(
        _mlp_kernel,
        grid=(NP // blk,),
        in_specs=[
            pl.BlockSpec((blk, H), lambda i: (i, 0)),
            pl.BlockSpec((H, H), lambda i: (0, 0)),
            pl.BlockSpec((1, H), lambda i: (0, 0)),
        ],
        out_specs=pl.BlockSpec((blk, H), lambda i: (i, 0)),
        out_shape=jax.ShapeDtypeStruct((NP, H), jnp.float32),
    )(pre, w2, b2.reshape(1, H))


def _iota16():
    return lax.broadcasted_iota(jnp.int32, (LANES,), 0)


def _sc_base_body(plut_hbm, olut_hbm, trig_hbm, act_hbm, targ_hbm, op_hbm,
                  out_hbm, plut_v, olut_v, t_v, a_v, g_v, o_v, obuf, sem):
    wid = lax.axis_index("s") * 2 + lax.axis_index("c")
    pltpu.sync_copy(plut_hbm, plut_v)
    pltpu.sync_copy(olut_hbm, olut_v)
    iota = _iota16()
    base_p = wid * (NP // NW)
    for k in range(NP // NW // PCH):
        p0 = pl.multiple_of(base_p + k * PCH, PCH)
        pltpu.async_copy(trig_hbm.at[pl.ds(p0, PCH)], t_v, sem)
        pltpu.async_copy(act_hbm.at[pl.ds(p0, PCH)], a_v, sem)
        pltpu.async_copy(targ_hbm.at[pl.ds(p0, PCH)], g_v, sem)
        cp = pltpu.async_copy(op_hbm.at[pl.ds(p0, PCH)], o_v, sem)
        cp.wait()
        cp.wait()
        cp.wait()
        cp.wait()

        def grp(i, carry):
            t16 = t_v[pl.ds(i * LANES, LANES)]
            a16 = a_v[pl.ds(i * LANES, LANES)]
            g16 = g_v[pl.ds(i * LANES, LANES)]
            o16 = o_v[pl.ds(i * LANES, LANES)]
            cb = (t16 * 90 + a16 * 10 + g16) * H
            ob = o16 * H
            lb = (i * LANES + iota) * H
            for j in range(H):
                v = plsc.load_gather(plut_v, [cb + j])
                v = v + plsc.load_gather(olut_v, [ob + j])
                plsc.store_scatter(obuf, [lb + j], v)
            return carry

        lax.fori_loop(0, PCH // LANES, grp, 0)
        pltpu.sync_copy(obuf, out_hbm.at[pl.ds(p0 * H, PCH * H)])


def _sc_base(plut, olut, trigger, action, target, operand_id):
    mesh = plsc.VectorSubcoreMesh(core_axis_name="c", subcore_axis_name="s",
                                  num_cores=2, num_subcores=16)
    fn = pl.kernel(
        _sc_base_body,
        out_type=jax.ShapeDtypeStruct((NP * H,), jnp.float32),
        mesh=mesh,
        scratch_types=[
            pltpu.VMEM((CPAD * H,), jnp.float32),
            pltpu.VMEM((16 * H,), jnp.float32),
            pltpu.VMEM((PCH,), jnp.int32),
            pltpu.VMEM((PCH,), jnp.int32),
            pltpu.VMEM((PCH,), jnp.int32),
            pltpu.VMEM((PCH,), jnp.int32),
            pltpu.VMEM((PCH * H,), jnp.float32),
            pltpu.SemaphoreType.DMA,
        ],
        compiler_params=pltpu.CompilerParams(needs_layout_passes=False),
    )
    return fn(plut.reshape(-1), olut.reshape(-1), trigger, action, target,
              operand_id)


def _sc_child_body(flut_hbm, ct_hbm, ca_hbm, cg_hbm, cp_hbm, starts_hbm,
                   pre0_hbm, out_hbm, flut_v, ct_v, ca_v, cg_v, cp_v, st_v,
                   acc, sem):
    wid = lax.axis_index("s") * 2 + lax.axis_index("c")
    pltpu.sync_copy(flut_hbm, flut_v)
    pltpu.sync_copy(starts_hbm, st_v)
    iota = _iota16()
    for kb in range(NBLK // NW):
        b = wid * (NBLK // NW) + kb
        p0 = pl.multiple_of(b * PB, PB)
        pltpu.sync_copy(pre0_hbm.at[pl.ds(p0 * H, PB * H)], acc)
        sv = st_v[pl.ds(b, LANES)]
        s = sv[0]
        e = sv[1]
        c0 = s & jnp.int32(~(LANES - 1))
        nch = (e - c0 + (CH - 1)) // CH

        def chunk(k, carry):
            cbase = pl.multiple_of(c0 + k * CH, LANES)
            pltpu.async_copy(ct_hbm.at[pl.ds(cbase, CH)], ct_v, sem)
            pltpu.async_copy(ca_hbm.at[pl.ds(cbase, CH)], ca_v, sem)
            pltpu.async_copy(cg_hbm.at[pl.ds(cbase, CH)], cg_v, sem)
            cp = pltpu.async_copy(cp_hbm.at[pl.ds(cbase, CH)], cp_v, sem)
            cp.wait()
            cp.wait()
            cp.wait()
            cp.wait()

            def grp(i, carry2):
                gidx = cbase + i * LANES + iota
                m = (gidx >= s) & (gidx < e)
                c16 = (ct_v[pl.ds(i * LANES, LANES)] * 90
                       + ca_v[pl.ds(i * LANES, LANES)] * 10
                       + cg_v[pl.ds(i * LANES, LANES)])
                pl16 = cp_v[pl.ds(i * LANES, LANES)] - p0
                cb = jnp.where(m, c16, 0) * H
                pb = jnp.where(m, pl16, 0) * H
                for j in range(H):
                    v = plsc.load_gather(flut_v, [cb + j], mask=m)
                    plsc.addupdate_scatter(acc, [pb + j], v, mask=m)
                return carry2

            lax.fori_loop(0, CH // LANES, grp, 0)
            return carry

        lax.fori_loop(0, nch, chunk, 0)
        pltpu.sync_copy(acc, out_hbm.at[pl.ds(p0 * H, PB * H)])


def _sc_child(flut, ct, ca, cg, cp, starts, pre0):
    mesh = plsc.VectorSubcoreMesh(core_axis_name="c", subcore_axis_name="s",
                                  num_cores=2, num_subcores=16)
    fn = pl.kernel(
        _sc_child_body,
        out_type=jax.ShapeDtypeStruct((NP * H,), jnp.float32),
        mesh=mesh,
        scratch_types=[
            pltpu.VMEM((CPAD * H,), jnp.float32),
            pltpu.VMEM((CH,), jnp.int32),
            pltpu.VMEM((CH,), jnp.int32),
            pltpu.VMEM((CH,), jnp.int32),
            pltpu.VMEM((CH,), jnp.int32),
            pltpu.VMEM((NBLK + LANES,), jnp.int32),
            pltpu.VMEM((PB * H,), jnp.float32),
            pltpu.SemaphoreType.DMA,
        ],
        compiler_params=pltpu.CompilerParams(needs_layout_passes=False),
    )
    return fn(flut.reshape(-1), ct, ca, cg, cp, starts, pre0)


def kernel(trigger, action, target, operand_id, child_trigger, child_action,
           child_target, child_parent, T_trigger, T_effect, T_target,
           T_operand, W_leaf, b_leaf, W_n1, b_n1, W_n2, b_n2):
    tt = T_trigger.at[0].set(0.0)
    te = T_effect.at[0].set(0.0)
    tg = T_target.at[0].set(0.0)
    to = T_operand.at[0].set(0.0)
    combo = (tt[:, None, None, :] + te[None, :, None, :]
             + tg[None, None, :, :]).reshape(NCOMBO, H)
    combo_pad = jnp.zeros((CPAD, H), jnp.float32).at[:NCOMBO].set(combo)
    to_pad = jnp.zeros((16, H), jnp.float32).at[:15].set(to)

    flut, plut, olut = _build_luts(combo_pad, W_leaf, b_leaf, W_n1, to_pad,
                                   b_n1)

    pre0 = _sc_base(plut, olut, trigger, action, target, operand_id)

    bounds = jnp.arange(0, NP + 1, PB, dtype=jnp.int32)
    starts = jnp.searchsorted(child_parent, bounds, side="left").astype(
        jnp.int32)
    starts_pad = jnp.zeros((NBLK + LANES,), jnp.int32).at[:NBLK + 1].set(
        starts)
    ct_pad = jnp.concatenate([child_trigger,
                              jnp.zeros((CH,), jnp.int32)])
    ca_pad = jnp.concatenate([child_action, jnp.zeros((CH,), jnp.int32)])
    cg_pad = jnp.concatenate([child_target, jnp.zeros((CH,), jnp.int32)])
    cp_pad = jnp.concatenate([child_parent, jnp.zeros((CH,), jnp.int32)])

    pre = _sc_child(flut, ct_pad, ca_pad, cg_pad, cp_pad, starts_pad, pre0)

    return _out_mlp(pre.reshape(NP, H), W_n2, b_n2)


# speedup vs baseline: 3.0857x; 1.2919x over previous
import functools

import jax
import jax.numpy as jnp
from jax import lax
from jax.experimental import pallas as pl
from jax.experimental.pallas import tpu as pltpu
from jax.experimental.pallas import tpu_sc as plsc

H = 96
NP = 65536
NC = 262144
NCOMBO = 630
CPAD = 640
NW = 32
PB = 512
NBLK = NP // PB
PCH = 512
CH = 1024
LANES = 16


def _lut_kernel(combo_ref, wl_ref, bl_ref, wn1_ref, to_ref, bn1_ref,
                flut_ref, plut_ref, olut_ref):
    combo = combo_ref[...]
    leaf = jnp.maximum(
        jnp.dot(combo, wl_ref[...], preferred_element_type=jnp.float32)
        + bl_ref[...], 0.0)
    wb = wn1_ref[H:, :]
    wa = wn1_ref[:H, :]
    flut_ref[...] = jnp.dot(leaf, wb, preferred_element_type=jnp.float32)
    plut_ref[...] = jnp.dot(combo, wb, preferred_element_type=jnp.float32)
    olut_ref[...] = (
        jnp.dot(to_ref[...], wa, preferred_element_type=jnp.float32)
        + bn1_ref[...])


def _build_luts(combo_pad, w_leaf, b_leaf, w_n1, to_pad, b_n1):
    return pl.pallas_call(
        _lut_kernel,
        out_shape=(
            jax.ShapeDtypeStruct((CPAD, H), jnp.float32),
            jax.ShapeDtypeStruct((CPAD, H), jnp.float32),
            jax.ShapeDtypeStruct((16, H), jnp.float32),
        ),
    )(combo_pad, w_leaf, b_leaf.reshape(1, H), w_n1, to_pad,
      b_n1.reshape(1, H))


def _mlp_kernel(pre_ref, w2_ref, b2_ref, out_ref):
    h = jnp.maximum(pre_ref[...], 0.0)
    out_ref[...] = (
        jnp.dot(h, w2_ref[...], preferred_element_type=jnp.float32)
        + b2_ref[...])


def _out_mlp(pre, w2, b2):
    blk = 4096
    return pl.pallas_call(
        _mlp_kernel,
        grid=(NP // blk,),
        in_specs=[
            pl.BlockSpec((blk, H), lambda i: (i, 0)),
            pl.BlockSpec((H, H), lambda i: (0, 0)),
            pl.BlockSpec((1, H), lambda i: (0, 0)),
        ],
        out_specs=pl.BlockSpec((blk, H), lambda i: (i, 0)),
        out_shape=jax.ShapeDtypeStruct((NP, H), jnp.float32),
    )(pre, w2, b2.reshape(1, H))


def _iota16():
    return lax.broadcasted_iota(jnp.int32, (LANES,), 0)


def _sc_base_body(plut_hbm, olut_hbm, trig_hbm, act_hbm, targ_hbm, op_hbm,
                  out_hbm, plut_v, olut_v, t_v, a_v, g_v, o_v, obuf, sem):
    wid = lax.axis_index("s") * 2 + lax.axis_index("c")
    pltpu.sync_copy(plut_hbm, plut_v)
    pltpu.sync_copy(olut_hbm, olut_v)
    iota = _iota16()
    base_p = wid * (NP // NW)

    def outer(k, carry_o):
        p0 = pl.multiple_of(base_p + k * PCH, PCH)
        pltpu.async_copy(trig_hbm.at[pl.ds(p0, PCH)], t_v, sem)
        pltpu.async_copy(act_hbm.at[pl.ds(p0, PCH)], a_v, sem)
        pltpu.async_copy(targ_hbm.at[pl.ds(p0, PCH)], g_v, sem)
        cp = pltpu.async_copy(op_hbm.at[pl.ds(p0, PCH)], o_v, sem)
        cp.wait()
        cp.wait()
        cp.wait()
        cp.wait()

        def grp(i, carry):
            t16 = t_v[pl.ds(i * LANES, LANES)]
            a16 = a_v[pl.ds(i * LANES, LANES)]
            g16 = g_v[pl.ds(i * LANES, LANES)]
            o16 = o_v[pl.ds(i * LANES, LANES)]
            cb = (t16 * 90 + a16 * 10 + g16) * H
            ob = o16 * H
            lb = (i * LANES + iota) * H
            for j0 in range(0, H, 8):
                vp = [plsc.load_gather(plut_v, [cb + (j0 + t)])
                      for t in range(8)]
                vo = [plsc.load_gather(olut_v, [ob + (j0 + t)])
                      for t in range(8)]
                for t in range(8):
                    plsc.store_scatter(obuf, [lb + (j0 + t)], vp[t] + vo[t])
            return carry

        lax.fori_loop(0, PCH // LANES, grp, 0)
        pltpu.sync_copy(obuf, out_hbm.at[pl.ds(p0 * H, PCH * H)])
        return carry_o

    lax.fori_loop(0, NP // NW // PCH, outer, 0)


def _sc_base(plut, olut, trigger, action, target, operand_id):
    mesh = plsc.VectorSubcoreMesh(core_axis_name="c", subcore_axis_name="s",
                                  num_cores=2, num_subcores=16)
    fn = pl.kernel(
        _sc_base_body,
        out_type=jax.ShapeDtypeStruct((NP * H,), jnp.float32),
        mesh=mesh,
        scratch_types=[
            pltpu.VMEM((CPAD * H,), jnp.float32),
            pltpu.VMEM((16 * H,), jnp.float32),
            pltpu.VMEM((PCH,), jnp.int32),
            pltpu.VMEM((PCH,), jnp.int32),
            pltpu.VMEM((PCH,), jnp.int32),
            pltpu.VMEM((PCH,), jnp.int32),
            pltpu.VMEM((PCH * H,), jnp.float32),
            pltpu.SemaphoreType.DMA,
        ],
        compiler_params=pltpu.CompilerParams(needs_layout_passes=False),
    )
    return fn(plut.reshape(-1), olut.reshape(-1), trigger, action, target,
              operand_id)


def _sc_child_body(flut_hbm, ct_hbm, ca_hbm, cg_hbm, cp_hbm, starts_hbm,
                   pre0_hbm, out_hbm, flut_v, ct_v, ca_v, cg_v, cp_v, st_v,
                   acc, sem):
    wid = lax.axis_index("s") * 2 + lax.axis_index("c")
    pltpu.sync_copy(flut_hbm, flut_v)
    pltpu.sync_copy(starts_hbm, st_v)
    iota = _iota16()
    def outer(kb, carry_o):
        b = wid * (NBLK // NW) + kb
        p0 = pl.multiple_of(b * PB, PB)
        pltpu.sync_copy(pre0_hbm.at[pl.ds(p0 * H, PB * H)], acc)
        sv = st_v[pl.ds(b, LANES)]
        s = sv[0]
        e = sv[1]
        c0 = s & jnp.int32(~(LANES - 1))
        nch = (e - c0 + (CH - 1)) // CH

        def chunk(k, carry):
            cbase = pl.multiple_of(c0 + k * CH, LANES)
            pltpu.async_copy(ct_hbm.at[pl.ds(cbase, CH)], ct_v, sem)
            pltpu.async_copy(ca_hbm.at[pl.ds(cbase, CH)], ca_v, sem)
            pltpu.async_copy(cg_hbm.at[pl.ds(cbase, CH)], cg_v, sem)
            cp = pltpu.async_copy(cp_hbm.at[pl.ds(cbase, CH)], cp_v, sem)
            cp.wait()
            cp.wait()
            cp.wait()
            cp.wait()

            def grp(i, carry2):
                gidx = cbase + i * LANES + iota
                m = (gidx >= s) & (gidx < e)
                c16 = (ct_v[pl.ds(i * LANES, LANES)] * 90
                       + ca_v[pl.ds(i * LANES, LANES)] * 10
                       + cg_v[pl.ds(i * LANES, LANES)])
                pl16 = cp_v[pl.ds(i * LANES, LANES)] - p0
                cb = jnp.where(m, c16, 0) * H
                pb = jnp.where(m, pl16, 0) * H
                for j0 in range(0, H, 8):
                    vs = [plsc.load_gather(flut_v, [cb + (j0 + t)], mask=m)
                          for t in range(8)]
                    for t in range(8):
                        plsc.addupdate_scatter(acc, [pb + (j0 + t)], vs[t],
                                               mask=m)
                return carry2

            lax.fori_loop(0, CH // LANES, grp, 0)
            return carry

        lax.fori_loop(0, nch, chunk, 0)
        pltpu.sync_copy(acc, out_hbm.at[pl.ds(p0 * H, PB * H)])
        return carry_o

    lax.fori_loop(0, NBLK // NW, outer, 0)


def _sc_child(flut, ct, ca, cg, cp, starts, pre0):
    mesh = plsc.VectorSubcoreMesh(core_axis_name="c", subcore_axis_name="s",
                                  num_cores=2, num_subcores=16)
    fn = pl.kernel(
        _sc_child_body,
        out_type=jax.ShapeDtypeStruct((NP * H,), jnp.float32),
        mesh=mesh,
        scratch_types=[
            pltpu.VMEM((CPAD * H,), jnp.float32),
            pltpu.VMEM((CH,), jnp.int32),
            pltpu.VMEM((CH,), jnp.int32),
            pltpu.VMEM((CH,), jnp.int32),
            pltpu.VMEM((CH,), jnp.int32),
            pltpu.VMEM((NBLK + LANES,), jnp.int32),
            pltpu.VMEM((PB * H,), jnp.float32),
            pltpu.SemaphoreType.DMA,
        ],
        compiler_params=pltpu.CompilerParams(needs_layout_passes=False),
    )
    return fn(flut.reshape(-1), ct, ca, cg, cp, starts, pre0)


def kernel(trigger, action, target, operand_id, child_trigger, child_action,
           child_target, child_parent, T_trigger, T_effect, T_target,
           T_operand, W_leaf, b_leaf, W_n1, b_n1, W_n2, b_n2):
    tt = T_trigger.at[0].set(0.0)
    te = T_effect.at[0].set(0.0)
    tg = T_target.at[0].set(0.0)
    to = T_operand.at[0].set(0.0)
    combo = (tt[:, None, None, :] + te[None, :, None, :]
             + tg[None, None, :, :]).reshape(NCOMBO, H)
    combo_pad = jnp.zeros((CPAD, H), jnp.float32).at[:NCOMBO].set(combo)
    to_pad = jnp.zeros((16, H), jnp.float32).at[:15].set(to)

    flut, plut, olut = _build_luts(combo_pad, W_leaf, b_leaf, W_n1, to_pad,
                                   b_n1)

    pre0 = _sc_base(plut, olut, trigger, action, target, operand_id)

    bounds = jnp.arange(0, NP + 1, PB, dtype=jnp.int32)
    starts = jnp.searchsorted(child_parent, bounds, side="left").astype(
        jnp.int32)
    starts_pad = jnp.zeros((NBLK + LANES,), jnp.int32).at[:NBLK + 1].set(
        starts)
    ct_pad = jnp.concatenate([child_trigger,
                              jnp.zeros((CH,), jnp.int32)])
    ca_pad = jnp.concatenate([child_action, jnp.zeros((CH,), jnp.int32)])
    cg_pad = jnp.concatenate([child_target, jnp.zeros((CH,), jnp.int32)])
    cp_pad = jnp.concatenate([child_parent, jnp.zeros((CH,), jnp.int32)])

    pre = _sc_child(flut, ct_pad, ca_pad, cg_pad, cp_pad, starts_pad, pre0)

    return _out_mlp(pre.reshape(NP, H), W_n2, b_n2)


# speedup vs baseline: 7.4541x; 2.4157x over previous
import jax
import jax.numpy as jnp
from jax import lax
from jax.experimental import pallas as pl
from jax.experimental.pallas import tpu as pltpu
from jax.experimental.pallas import tpu_sc as plsc

H = 96
HP = 97
NP = 65536
NC = 262144
NCOMBO = 630
CPAD = 640
NW = 32
PB = 512
NBLK = NP // PB
CH = 1024
LANES = 16
MBLK = 4096


def _dotf(x, w):
    return jnp.dot(x, w, preferred_element_type=jnp.float32,
                   precision=lax.Precision.HIGHEST)


def _lut_kernel(combo_ref, wl_ref, bl_ref, wn1_ref, tt_ref, te_ref, tg_ref,
                to_ref, bn1_ref, flut_ref, tw_ref, aw_ref, gw_ref, olut_ref):
    combo = combo_ref[...]
    leaf = jnp.maximum(_dotf(combo, wl_ref[...]) + bl_ref[...], 0.0)
    wb = wn1_ref[H:, :]
    wa = wn1_ref[:H, :]
    flut_ref[...] = _dotf(leaf, wb)
    tw_ref[...] = _dotf(tt_ref[...], wb)
    aw_ref[...] = _dotf(te_ref[...], wb)
    gw_ref[...] = _dotf(tg_ref[...], wb)
    olut_ref[...] = _dotf(to_ref[...], wa) + bn1_ref[...]


def _build_luts(combo_pad, w_leaf, b_leaf, w_n1, tt_pad, te_pad, tg_pad,
                to_pad, b_n1):
    return pl.pallas_call(
        _lut_kernel,
        out_shape=(
            jax.ShapeDtypeStruct((CPAD, H), jnp.float32),
            jax.ShapeDtypeStruct((8, H), jnp.float32),
            jax.ShapeDtypeStruct((16, H), jnp.float32),
            jax.ShapeDtypeStruct((16, H), jnp.float32),
            jax.ShapeDtypeStruct((16, H), jnp.float32),
        ),
    )(combo_pad, w_leaf, b_leaf.reshape(1, H), w_n1, tt_pad, te_pad, tg_pad,
      to_pad, b_n1.reshape(1, H))


def _onehot(idx, n):
    return (idx[:, None] == lax.broadcasted_iota(jnp.int32, (1, n), 1)
            ).astype(jnp.float32)


def _mlp_kernel(cs_ref, t_ref, a_ref, g_ref, o_ref, tw_ref, aw_ref, gw_ref,
                olut_ref, w2_ref, b2_ref, out_ref):
    dotd = lambda x, w: jnp.dot(x, w, preferred_element_type=jnp.float32)
    pre = (cs_ref[:, :H]
           + dotd(_onehot(t_ref[0, 0, :], 8), tw_ref[...])
           + dotd(_onehot(a_ref[0, 0, :], 16), aw_ref[...])
           + dotd(_onehot(g_ref[0, 0, :], 16), gw_ref[...])
           + dotd(_onehot(o_ref[0, 0, :], 16), olut_ref[...]))
    h = jnp.maximum(pre, 0.0)
    out_ref[...] = _dotf(h, w2_ref[...]) + b2_ref[...]


def _out_mlp(child_sum, trigger, action, target, operand_id, tw, aw, gw,
             olut, w2, b2):
    nb = NP // MBLK
    idx3 = lambda x: x.reshape(nb, 1, MBLK)
    ispec = pl.BlockSpec((1, 1, MBLK), lambda i: (i, 0, 0))
    return pl.pallas_call(
        _mlp_kernel,
        grid=(nb,),
        in_specs=[
            pl.BlockSpec((MBLK, HP), lambda i: (i, 0)),
            ispec, ispec, ispec, ispec,
            pl.BlockSpec((8, H), lambda i: (0, 0)),
            pl.BlockSpec((16, H), lambda i: (0, 0)),
            pl.BlockSpec((16, H), lambda i: (0, 0)),
            pl.BlockSpec((16, H), lambda i: (0, 0)),
            pl.BlockSpec((H, H), lambda i: (0, 0)),
            pl.BlockSpec((1, H), lambda i: (0, 0)),
        ],
        out_specs=pl.BlockSpec((MBLK, H), lambda i: (i, 0)),
        out_shape=jax.ShapeDtypeStruct((NP, H), jnp.float32),
    )(child_sum, idx3(trigger), idx3(action), idx3(target), idx3(operand_id),
      tw, aw, gw, olut, w2, b2.reshape(1, H))


def _iota16():
    return lax.broadcasted_iota(jnp.int32, (LANES,), 0)


def _sc_child_body(flut_hbm, ct_hbm, ca_hbm, cg_hbm, cp_hbm, starts_hbm,
                   out_hbm, flut_v, ct_v, ca_v, cg_v, cp_v, st_v, acc, sem):
    wid = lax.axis_index("s") * 2 + lax.axis_index("c")
    pltpu.sync_copy(flut_hbm, flut_v)
    pltpu.sync_copy(starts_hbm, st_v)
    iota = _iota16()

    def outer(kb, carry_o):
        b = wid * (NBLK // NW) + kb
        p0 = pl.multiple_of(b * PB, PB)

        def zloop(z, carry_z):
            base = z * (8 * LANES)
            for u in range(8):
                acc[pl.ds(base + u * LANES, LANES)] = jnp.zeros(
                    (LANES,), jnp.float32)
            return carry_z

        lax.fori_loop(0, PB * HP // (8 * LANES), zloop, 0)

        sv = st_v[pl.ds(b, LANES)]
        s = sv[0]
        e = sv[1]
        c0 = s & jnp.int32(~(LANES - 1))
        nch = (e - c0 + (CH - 1)) // CH

        def chunk(k, carry):
            cbase = pl.multiple_of(c0 + k * CH, LANES)
            pltpu.async_copy(ct_hbm.at[pl.ds(cbase, CH)], ct_v, sem)
            pltpu.async_copy(ca_hbm.at[pl.ds(cbase, CH)], ca_v, sem)
            pltpu.async_copy(cg_hbm.at[pl.ds(cbase, CH)], cg_v, sem)
            cp = pltpu.async_copy(cp_hbm.at[pl.ds(cbase, CH)], cp_v, sem)
            cp.wait()
            cp.wait()
            cp.wait()
            cp.wait()

            def grp(i, carry2):
                gidx = cbase + i * LANES + iota
                m = (gidx >= s) & (gidx < e)
                c16 = (ct_v[pl.ds(i * LANES, LANES)] * 90
                       + ca_v[pl.ds(i * LANES, LANES)] * 10
                       + cg_v[pl.ds(i * LANES, LANES)])
                pl16 = cp_v[pl.ds(i * LANES, LANES)] - p0
                cb = jnp.where(m, c16, 0) * HP
                pb = jnp.where(m, pl16, 0) * HP
                for j0 in range(0, H, 8):
                    vs = [plsc.load_gather(flut_v, [cb + (j0 + t)], mask=m)
                          for t in range(8)]
                    for t in range(8):
                        plsc.addupdate_scatter(acc, [pb + (j0 + t)], vs[t],
                                               mask=m)
                return carry2

            ngrp = jnp.minimum((e - cbase + (LANES - 1)) // LANES,
                               CH // LANES)
            lax.fori_loop(0, jnp.maximum(ngrp, 0), grp, 0)
            return carry

        lax.fori_loop(0, nch, chunk, 0)
        pltpu.sync_copy(acc, out_hbm.at[pl.ds(p0 * HP, PB * HP)])
        return carry_o

    lax.fori_loop(0, NBLK // NW, outer, 0)


def _sc_child(flut_pad, ct, ca, cg, cp, starts):
    mesh = plsc.VectorSubcoreMesh(core_axis_name="c", subcore_axis_name="s",
                                  num_cores=2, num_subcores=16)
    fn = pl.kernel(
        _sc_child_body,
        out_type=jax.ShapeDtypeStruct((NP * HP,), jnp.float32),
        mesh=mesh,
        scratch_types=[
            pltpu.VMEM((CPAD * HP,), jnp.float32),
            pltpu.VMEM((CH,), jnp.int32),
            pltpu.VMEM((CH,), jnp.int32),
            pltpu.VMEM((CH,), jnp.int32),
            pltpu.VMEM((CH,), jnp.int32),
            pltpu.VMEM((NBLK + LANES,), jnp.int32),
            pltpu.VMEM((PB * HP,), jnp.float32),
            pltpu.SemaphoreType.DMA,
        ],
        compiler_params=pltpu.CompilerParams(needs_layout_passes=False),
    )
    return fn(flut_pad, ct, ca, cg, cp, starts)


def kernel(trigger, action, target, operand_id, child_trigger, child_action,
           child_target, child_parent, T_trigger, T_effect, T_target,
           T_operand, W_leaf, b_leaf, W_n1, b_n1, W_n2, b_n2):
    tt = T_trigger.at[0].set(0.0)
    te = T_effect.at[0].set(0.0)
    tg = T_target.at[0].set(0.0)
    to = T_operand.at[0].set(0.0)
    combo = (tt[:, None, None, :] + te[None, :, None, :]
             + tg[None, None, :, :]).reshape(NCOMBO, H)
    combo_pad = jnp.zeros((CPAD, H), jnp.float32).at[:NCOMBO].set(combo)
    to_pad = jnp.zeros((16, H), jnp.float32).at[:15].set(to)
    tt_pad = jnp.zeros((8, H), jnp.float32).at[:7].set(tt)
    te_pad = jnp.zeros((16, H), jnp.float32).at[:9].set(te)
    tg_pad = jnp.zeros((16, H), jnp.float32).at[:10].set(tg)

    flut, tw, aw, gw, olut = _build_luts(combo_pad, W_leaf, b_leaf, W_n1,
                                         tt_pad, te_pad, tg_pad, to_pad, b_n1)
    flut_pad = jnp.zeros((CPAD, HP), jnp.float32).at[:, :H].set(flut)

    bounds = jnp.arange(0, NP + 1, PB, dtype=jnp.int32)
    starts = jnp.searchsorted(child_parent, bounds, side="left").astype(
        jnp.int32)
    starts_pad = jnp.zeros((NBLK + LANES,), jnp.int32).at[:NBLK + 1].set(
        starts)
    ct_pad = jnp.concatenate([child_trigger, jnp.zeros((CH,), jnp.int32)])
    ca_pad = jnp.concatenate([child_action, jnp.zeros((CH,), jnp.int32)])
    cg_pad = jnp.concatenate([child_target, jnp.zeros((CH,), jnp.int32)])
    cp_pad = jnp.concatenate([child_parent, jnp.zeros((CH,), jnp.int32)])

    child_sum = _sc_child(flut_pad.reshape(-1), ct_pad, ca_pad, cg_pad,
                          cp_pad, starts_pad)

    return _out_mlp(child_sum.reshape(NP, HP), trigger, action, target,
                    operand_id, tw, aw, gw, olut, W_n2, b_n2)


# speedup vs baseline: 13.0043x; 1.7446x over previous
import jax
import jax.numpy as jnp
from jax import lax
from jax.experimental import pallas as pl
from jax.experimental.pallas import tpu as pltpu
from jax.experimental.pallas import tpu_sc as plsc

H = 96
HP = 97
NP = 65536
NC = 262144
NCOMBO = 630
CPAD = 640
NW = 32
PB = 512
NBLK = NP // PB
CH = 1024
LANES = 16
MBLK = 4096


def _dotf(x, w):
    return jnp.dot(x, w, preferred_element_type=jnp.float32,
                   precision=lax.Precision.HIGHEST)


def _lut_kernel(combo_ref, wl_ref, bl_ref, wn1_ref, tt_ref, te_ref, tg_ref,
                to_ref, bn1_ref, flut_ref, tw_ref, aw_ref, gw_ref, olut_ref):
    combo = combo_ref[...]
    leaf = jnp.maximum(_dotf(combo, wl_ref[...]) + bl_ref[...], 0.0)
    wb = wn1_ref[H:, :]
    wa = wn1_ref[:H, :]
    flut_ref[...] = _dotf(leaf, wb)
    tw_ref[...] = _dotf(tt_ref[...], wb)
    aw_ref[...] = _dotf(te_ref[...], wb)
    gw_ref[...] = _dotf(tg_ref[...], wb)
    olut_ref[...] = _dotf(to_ref[...], wa) + bn1_ref[...]


def _build_luts(combo_pad, w_leaf, b_leaf, w_n1, tt_pad, te_pad, tg_pad,
                to_pad, b_n1):
    return pl.pallas_call(
        _lut_kernel,
        out_shape=(
            jax.ShapeDtypeStruct((CPAD, H), jnp.float32),
            jax.ShapeDtypeStruct((8, H), jnp.float32),
            jax.ShapeDtypeStruct((16, H), jnp.float32),
            jax.ShapeDtypeStruct((16, H), jnp.float32),
            jax.ShapeDtypeStruct((16, H), jnp.float32),
        ),
    )(combo_pad, w_leaf, b_leaf.reshape(1, H), w_n1, tt_pad, te_pad, tg_pad,
      to_pad, b_n1.reshape(1, H))


def _onehot(idx, n):
    return (idx[:, None] == lax.broadcasted_iota(jnp.int32, (1, n), 1)
            ).astype(jnp.float32)


def _mlp_kernel(cs_ref, t_ref, a_ref, g_ref, o_ref, tw_ref, aw_ref, gw_ref,
                olut_ref, w2_ref, b2_ref, out_ref):
    dotd = lambda x, w: jnp.dot(x, w, preferred_element_type=jnp.float32)
    pre = (cs_ref[:, :H]
           + dotd(_onehot(t_ref[0, 0, :], 8), tw_ref[...])
           + dotd(_onehot(a_ref[0, 0, :], 16), aw_ref[...])
           + dotd(_onehot(g_ref[0, 0, :], 16), gw_ref[...])
           + dotd(_onehot(o_ref[0, 0, :], 16), olut_ref[...]))
    h = jnp.maximum(pre, 0.0)
    out_ref[...] = _dotf(h, w2_ref[...]) + b2_ref[...]


def _out_mlp(child_sum, trigger, action, target, operand_id, tw, aw, gw,
             olut, w2, b2):
    nb = NP // MBLK
    idx3 = lambda x: x.reshape(nb, 1, MBLK)
    ispec = pl.BlockSpec((1, 1, MBLK), lambda i: (i, 0, 0))
    return pl.pallas_call(
        _mlp_kernel,
        grid=(nb,),
        in_specs=[
            pl.BlockSpec((MBLK, HP), lambda i: (i, 0)),
            ispec, ispec, ispec, ispec,
            pl.BlockSpec((8, H), lambda i: (0, 0)),
            pl.BlockSpec((16, H), lambda i: (0, 0)),
            pl.BlockSpec((16, H), lambda i: (0, 0)),
            pl.BlockSpec((16, H), lambda i: (0, 0)),
            pl.BlockSpec((H, H), lambda i: (0, 0)),
            pl.BlockSpec((1, H), lambda i: (0, 0)),
        ],
        out_specs=pl.BlockSpec((MBLK, H), lambda i: (i, 0)),
        out_shape=jax.ShapeDtypeStruct((NP, H), jnp.float32),
    )(child_sum, idx3(trigger), idx3(action), idx3(target), idx3(operand_id),
      tw, aw, gw, olut, w2, b2.reshape(1, H))


def _iota16():
    return lax.broadcasted_iota(jnp.int32, (LANES,), 0)


def _sc_child_body(flut_hbm, ct_hbm, ca_hbm, cg_hbm, cp_hbm, starts_hbm,
                   out_hbm, flut_v, ct_v, ca_v, cg_v, cp_v, st_v, acc, sem):
    wid = lax.axis_index("s") * 2 + lax.axis_index("c")
    pltpu.sync_copy(flut_hbm, flut_v)
    pltpu.sync_copy(starts_hbm, st_v)
    iota = _iota16()

    def outer(kb, carry_o):
        b = wid * (NBLK // NW) + kb
        p0 = pl.multiple_of(b * PB, PB)

        def zloop(z, carry_z):
            base = z * (8 * LANES)
            for u in range(8):
                acc[pl.ds(base + u * LANES, LANES)] = jnp.zeros(
                    (LANES,), jnp.float32)
            return carry_z

        lax.fori_loop(0, PB * HP // (8 * LANES), zloop, 0)

        sv = st_v[pl.ds(b, LANES)]
        s = sv[0]
        e = sv[1]
        c0 = s & jnp.int32(~(LANES - 1))
        nch = (e - c0 + (CH - 1)) // CH
        zreg = jnp.zeros((LANES,), jnp.float32)

        def chunk(k, carry):
            cbase = pl.multiple_of(c0 + k * CH, LANES)
            pltpu.async_copy(ct_hbm.at[pl.ds(cbase, CH)], ct_v, sem)
            pltpu.async_copy(ca_hbm.at[pl.ds(cbase, CH)], ca_v, sem)
            pltpu.async_copy(cg_hbm.at[pl.ds(cbase, CH)], cg_v, sem)
            cp = pltpu.async_copy(cp_hbm.at[pl.ds(cbase, CH)], cp_v, sem)
            cp.wait()
            cp.wait()
            cp.wait()
            cp.wait()

            def grp(i, carry2):
                prev = carry2[0]
                regs = list(carry2[1:])
                pl16 = cp_v[pl.ds(i * LANES, LANES)] - p0
                bad = (pl16 < 0) | (pl16 >= PB)
                pb16 = jnp.where(bad, PB, pl16) * HP
                cb16 = (ct_v[pl.ds(i * LANES, LANES)] * 90
                        + ca_v[pl.ds(i * LANES, LANES)] * 10
                        + cg_v[pl.ds(i * LANES, LANES)]) * HP
                for k2 in range(LANES):
                    ck = cb16[k2]
                    pk = pb16[k2]
                    fresh = pk != prev
                    vj = [flut_v[pl.ds(ck + 16 * j, LANES)]
                          for j in range(6)]
                    regs = [jnp.where(fresh, vj[j], regs[j] + vj[j])
                            for j in range(6)]
                    for j in range(6):
                        acc[pl.ds(pk + 16 * j, LANES)] = regs[j]
                    prev = pk
                return (prev, regs[0], regs[1], regs[2], regs[3], regs[4],
                        regs[5])

            ngrp = jnp.minimum((e - cbase + (LANES - 1)) // LANES,
                               CH // LANES)
            return lax.fori_loop(0, jnp.maximum(ngrp, 0), grp, carry)

        carry0 = (jnp.int32(PB * HP), zreg, zreg, zreg, zreg, zreg, zreg)
        lax.fori_loop(0, nch, chunk, carry0)
        pltpu.sync_copy(acc.at[pl.ds(0, PB * HP)],
                        out_hbm.at[pl.ds(p0 * HP, PB * HP)])
        return carry_o

    lax.fori_loop(0, NBLK // NW, outer, 0)


def _sc_child(flut_pad, ct, ca, cg, cp, starts):
    mesh = plsc.VectorSubcoreMesh(core_axis_name="c", subcore_axis_name="s",
                                  num_cores=2, num_subcores=16)
    fn = pl.kernel(
        _sc_child_body,
        out_type=jax.ShapeDtypeStruct((NP * HP,), jnp.float32),
        mesh=mesh,
        scratch_types=[
            pltpu.VMEM((CPAD * HP,), jnp.float32),
            pltpu.VMEM((CH,), jnp.int32),
            pltpu.VMEM((CH,), jnp.int32),
            pltpu.VMEM((CH,), jnp.int32),
            pltpu.VMEM((CH,), jnp.int32),
            pltpu.VMEM((NBLK + LANES,), jnp.int32),
            pltpu.VMEM(((PB + 1) * HP,), jnp.float32),
            pltpu.SemaphoreType.DMA,
        ],
        compiler_params=pltpu.CompilerParams(needs_layout_passes=False),
    )
    return fn(flut_pad, ct, ca, cg, cp, starts)


def kernel(trigger, action, target, operand_id, child_trigger, child_action,
           child_target, child_parent, T_trigger, T_effect, T_target,
           T_operand, W_leaf, b_leaf, W_n1, b_n1, W_n2, b_n2):
    tt = T_trigger.at[0].set(0.0)
    te = T_effect.at[0].set(0.0)
    tg = T_target.at[0].set(0.0)
    to = T_operand.at[0].set(0.0)
    combo = (tt[:, None, None, :] + te[None, :, None, :]
             + tg[None, None, :, :]).reshape(NCOMBO, H)
    combo_pad = jnp.zeros((CPAD, H), jnp.float32).at[:NCOMBO].set(combo)
    to_pad = jnp.zeros((16, H), jnp.float32).at[:15].set(to)
    tt_pad = jnp.zeros((8, H), jnp.float32).at[:7].set(tt)
    te_pad = jnp.zeros((16, H), jnp.float32).at[:9].set(te)
    tg_pad = jnp.zeros((16, H), jnp.float32).at[:10].set(tg)

    flut, tw, aw, gw, olut = _build_luts(combo_pad, W_leaf, b_leaf, W_n1,
                                         tt_pad, te_pad, tg_pad, to_pad, b_n1)
    flut_pad = jnp.zeros((CPAD, HP), jnp.float32).at[:, :H].set(flut)

    bounds = jnp.arange(0, NP + 1, PB, dtype=jnp.int32)
    starts = jnp.searchsorted(child_parent, bounds, side="left",
                              method="compare_all").astype(jnp.int32)
    starts_pad = jnp.zeros((NBLK + LANES,), jnp.int32).at[:NBLK + 1].set(
        starts)
    ct_pad = jnp.concatenate([child_trigger, jnp.zeros((CH,), jnp.int32)])
    ca_pad = jnp.concatenate([child_action, jnp.zeros((CH,), jnp.int32)])
    cg_pad = jnp.concatenate([child_target, jnp.zeros((CH,), jnp.int32)])
    cp_pad = jnp.concatenate([child_parent,
                              jnp.full((CH,), NP, jnp.int32)])

    child_sum = _sc_child(flut_pad.reshape(-1), ct_pad, ca_pad, cg_pad,
                          cp_pad, starts_pad)

    return _out_mlp(child_sum.reshape(NP, HP), trigger, action, target,
                    operand_id, tw, aw, gw, olut, W_n2, b_n2)


# speedup vs baseline: 15.6749x; 1.2054x over previous
import jax
import jax.numpy as jnp
from jax import lax
from jax.experimental import pallas as pl
from jax.experimental.pallas import tpu as pltpu
from jax.experimental.pallas import tpu_sc as plsc

H = 96
AW = 128
NP = 65536
NC = 262144
NCOMBO = 630
CPAD = 640
NW = 32
PB = 512
NBLK = NP // PB
CH = 768
LANES = 16
MBLK = 4096


def _dotf(x, w):
    return jnp.dot(x, w, preferred_element_type=jnp.float32,
                   precision=lax.Precision.HIGHEST)


def _lut_kernel(combo_ref, wl_ref, bl_ref, wn1_ref, tt_ref, te_ref, tg_ref,
                to_ref, bn1_ref, flut_ref, tw_ref, aw_ref, gw_ref, olut_ref):
    combo = combo_ref[...]
    leaf = jnp.maximum(_dotf(combo, wl_ref[...]) + bl_ref[...], 0.0)
    wb = wn1_ref[H:, :]
    wa = wn1_ref[:H, :]
    flut_ref[...] = _dotf(leaf, wb)
    tw_ref[...] = _dotf(tt_ref[...], wb)
    aw_ref[...] = _dotf(te_ref[...], wb)
    gw_ref[...] = _dotf(tg_ref[...], wb)
    olut_ref[...] = _dotf(to_ref[...], wa) + bn1_ref[...]


def _build_luts(combo_pad, w_leaf, b_leaf, w_n1, tt_pad, te_pad, tg_pad,
                to_pad, b_n1):
    return pl.pallas_call(
        _lut_kernel,
        out_shape=(
            jax.ShapeDtypeStruct((CPAD, H), jnp.float32),
            jax.ShapeDtypeStruct((8, H), jnp.float32),
            jax.ShapeDtypeStruct((16, H), jnp.float32),
            jax.ShapeDtypeStruct((16, H), jnp.float32),
            jax.ShapeDtypeStruct((16, H), jnp.float32),
        ),
    )(combo_pad, w_leaf, b_leaf.reshape(1, H), w_n1, tt_pad, te_pad, tg_pad,
      to_pad, b_n1.reshape(1, H))


def _onehot(idx, n):
    return (idx[:, None] == lax.broadcasted_iota(jnp.int32, (1, n), 1)
            ).astype(jnp.float32)


def _mlp_kernel(cs_ref, t_ref, a_ref, g_ref, o_ref, tw_ref, aw_ref, gw_ref,
                olut_ref, w2_ref, b2_ref, out_ref):
    dotd = lambda x, w: jnp.dot(x, w, preferred_element_type=jnp.float32)
    pre = (cs_ref[:, :H]
           + dotd(_onehot(t_ref[0, 0, :], 8), tw_ref[...])
           + dotd(_onehot(a_ref[0, 0, :], 16), aw_ref[...])
           + dotd(_onehot(g_ref[0, 0, :], 16), gw_ref[...])
           + dotd(_onehot(o_ref[0, 0, :], 16), olut_ref[...]))
    h = jnp.maximum(pre, 0.0)
    out_ref[...] = _dotf(h, w2_ref[...]) + b2_ref[...]


def _out_mlp(child_sum, trigger, action, target, operand_id, tw, aw, gw,
             olut, w2, b2):
    nb = NP // MBLK
    idx3 = lambda x: x.reshape(nb, 1, MBLK)
    ispec = pl.BlockSpec((1, 1, MBLK), lambda i: (i, 0, 0))
    return pl.pallas_call(
        _mlp_kernel,
        grid=(nb,),
        in_specs=[
            pl.BlockSpec((MBLK, AW), lambda i: (i, 0)),
            ispec, ispec, ispec, ispec,
            pl.BlockSpec((8, H), lambda i: (0, 0)),
            pl.BlockSpec((16, H), lambda i: (0, 0)),
            pl.BlockSpec((16, H), lambda i: (0, 0)),
            pl.BlockSpec((16, H), lambda i: (0, 0)),
            pl.BlockSpec((H, H), lambda i: (0, 0)),
            pl.BlockSpec((1, H), lambda i: (0, 0)),
        ],
        out_specs=pl.BlockSpec((MBLK, H), lambda i: (i, 0)),
        out_shape=jax.ShapeDtypeStruct((NP, H), jnp.float32),
    )(child_sum, idx3(trigger), idx3(action), idx3(target), idx3(operand_id),
      tw, aw, gw, olut, w2, b2.reshape(1, H))


def _iota16():
    return lax.broadcasted_iota(jnp.int32, (LANES,), 0)


def _sc_child_body(flut_hbm, ct_hbm, ca_hbm, cg_hbm, cp_hbm, starts_hbm,
                   out_hbm, flut_v, ct_v, ca_v, cg_v, cp_v, st_v, acc, sem):
    wid = lax.axis_index("s") * 2 + lax.axis_index("c")
    pltpu.sync_copy(flut_hbm, flut_v)
    pltpu.sync_copy(starts_hbm, st_v)
    iota = _iota16()

    def outer(kb, carry_o):
        b = wid * (NBLK // NW) + kb
        p0 = pl.multiple_of(b * PB, PB)

        def zloop(z, carry_z):
            base = z * AW
            for u in range(6):
                acc[pl.ds(base + u * LANES, LANES)] = jnp.zeros(
                    (LANES,), jnp.float32)
            return carry_z

        lax.fori_loop(0, PB, zloop, 0)

        sv = st_v[pl.ds(b, LANES)]
        s = sv[0]
        e = sv[1]
        c0 = s & jnp.int32(~(LANES - 1))
        nch = (e - c0 + (CH - 1)) // CH
        zreg = jnp.zeros((LANES,), jnp.float32)

        def chunk(k, carry):
            cbase = pl.multiple_of(c0 + k * CH, LANES)
            pltpu.async_copy(ct_hbm.at[pl.ds(cbase, CH)], ct_v, sem)
            pltpu.async_copy(ca_hbm.at[pl.ds(cbase, CH)], ca_v, sem)
            pltpu.async_copy(cg_hbm.at[pl.ds(cbase, CH)], cg_v, sem)
            cp = pltpu.async_copy(cp_hbm.at[pl.ds(cbase, CH)], cp_v, sem)
            cp.wait()
            cp.wait()
            cp.wait()
            cp.wait()

            def grp(i, carry2):
                prev = carry2[0]
                regs = list(carry2[1:])
                pl16 = cp_v[pl.ds(i * LANES, LANES)] - p0
                bad = (pl16 < 0) | (pl16 >= PB)
                pb16 = jnp.where(bad, PB, pl16) * AW
                cb16 = (ct_v[pl.ds(i * LANES, LANES)] * 90
                        + ca_v[pl.ds(i * LANES, LANES)] * 10
                        + cg_v[pl.ds(i * LANES, LANES)]) * H
                for k2 in range(LANES):
                    ck = cb16[k2]
                    pk = pb16[k2]
                    fresh = pk != prev
                    vj = [flut_v[pl.ds(ck + 16 * j, LANES)]
                          for j in range(6)]
                    regs = [jnp.where(fresh, vj[j], regs[j] + vj[j])
                            for j in range(6)]
                    for j in range(6):
                        acc[pl.ds(pk + 16 * j, LANES)] = regs[j]
                    prev = pk
                return (prev, regs[0], regs[1], regs[2], regs[3], regs[4],
                        regs[5])

            ngrp = jnp.minimum((e - cbase + (LANES - 1)) // LANES,
                               CH // LANES)
            return lax.fori_loop(0, jnp.maximum(ngrp, 0), grp, carry)

        carry0 = (jnp.int32(PB * AW), zreg, zreg, zreg, zreg, zreg, zreg)
        lax.fori_loop(0, nch, chunk, carry0)
        pltpu.sync_copy(acc.at[pl.ds(0, PB * AW)],
                        out_hbm.at[pl.ds(p0 * AW, PB * AW)])
        return carry_o

    lax.fori_loop(0, NBLK // NW, outer, 0)


def _sc_child(flut_pad, ct, ca, cg, cp, starts):
    mesh = plsc.VectorSubcoreMesh(core_axis_name="c", subcore_axis_name="s",
                                  num_cores=2, num_subcores=16)
    fn = pl.kernel(
        _sc_child_body,
        out_type=jax.ShapeDtypeStruct((NP * AW,), jnp.float32),
        mesh=mesh,
        scratch_types=[
            pltpu.VMEM((CPAD * H,), jnp.float32),
            pltpu.VMEM((CH,), jnp.int32),
            pltpu.VMEM((CH,), jnp.int32),
            pltpu.VMEM((CH,), jnp.int32),
            pltpu.VMEM((CH,), jnp.int32),
            pltpu.VMEM((NBLK + LANES,), jnp.int32),
            pltpu.VMEM(((PB + 1) * AW,), jnp.float32),
            pltpu.SemaphoreType.DMA,
        ],
        compiler_params=pltpu.CompilerParams(needs_layout_passes=False),
    )
    return fn(flut_pad, ct, ca, cg, cp, starts)


def kernel(trigger, action, target, operand_id, child_trigger, child_action,
           child_target, child_parent, T_trigger, T_effect, T_target,
           T_operand, W_leaf, b_leaf, W_n1, b_n1, W_n2, b_n2):
    tt = T_trigger.at[0].set(0.0)
    te = T_effect.at[0].set(0.0)
    tg = T_target.at[0].set(0.0)
    to = T_operand.at[0].set(0.0)
    combo = (tt[:, None, None, :] + te[None, :, None, :]
             + tg[None, None, :, :]).reshape(NCOMBO, H)
    combo_pad = jnp.zeros((CPAD, H), jnp.float32).at[:NCOMBO].set(combo)
    to_pad = jnp.zeros((16, H), jnp.float32).at[:15].set(to)
    tt_pad = jnp.zeros((8, H), jnp.float32).at[:7].set(tt)
    te_pad = jnp.zeros((16, H), jnp.float32).at[:9].set(te)
    tg_pad = jnp.zeros((16, H), jnp.float32).at[:10].set(tg)

    flut, tw, aw, gw, olut = _build_luts(combo_pad, W_leaf, b_leaf, W_n1,
                                         tt_pad, te_pad, tg_pad, to_pad, b_n1)

    bounds = jnp.arange(0, NP + 1, PB, dtype=jnp.int32)
    starts = jnp.searchsorted(child_parent, bounds, side="left",
                              method="compare_all").astype(jnp.int32)
    starts_pad = jnp.zeros((NBLK + LANES,), jnp.int32).at[:NBLK + 1].set(
        starts)
    ct_pad = jnp.concatenate([child_trigger, jnp.zeros((CH,), jnp.int32)])
    ca_pad = jnp.concatenate([child_action, jnp.zeros((CH,), jnp.int32)])
    cg_pad = jnp.concatenate([child_target, jnp.zeros((CH,), jnp.int32)])
    cp_pad = jnp.concatenate([child_parent,
                              jnp.full((CH,), NP, jnp.int32)])

    child_sum = _sc_child(flut.reshape(-1), ct_pad, ca_pad, cg_pad,
                          cp_pad, starts_pad)

    return _out_mlp(child_sum.reshape(NP, AW), trigger, action, target,
                    operand_id, tw, aw, gw, olut, W_n2, b_n2)


# speedup vs baseline: 16.1774x; 1.0321x over previous
import jax
import jax.numpy as jnp
from jax import lax
from jax.experimental import pallas as pl
from jax.experimental.pallas import tpu as pltpu
from jax.experimental.pallas import tpu_sc as plsc

H = 96
AW = 128
NP = 65536
NC = 262144
NCOMBO = 630
CPAD = 640
NW = 32
PB = 512
NBLK = NP // PB
CH = 384
LANES = 16
MBLK = 8192


def _dotf(x, w):
    return jnp.dot(x, w, preferred_element_type=jnp.float32,
                   precision=lax.Precision.HIGHEST)


def _lut_kernel(combo_ref, wl_ref, bl_ref, wn1_ref, tt_ref, te_ref, tg_ref,
                to_ref, bn1_ref, flut_ref, tw_ref, aw_ref, gw_ref, olut_ref):
    combo = combo_ref[...]
    leaf = jnp.maximum(_dotf(combo, wl_ref[...]) + bl_ref[...], 0.0)
    wb = wn1_ref[H:, :]
    wa = wn1_ref[:H, :]
    flut_ref[...] = _dotf(leaf, wb)
    tw_ref[...] = _dotf(tt_ref[...], wb)
    aw_ref[...] = _dotf(te_ref[...], wb)
    gw_ref[...] = _dotf(tg_ref[...], wb)
    olut_ref[...] = _dotf(to_ref[...], wa) + bn1_ref[...]


def _build_luts(combo_pad, w_leaf, b_leaf, w_n1, tt_pad, te_pad, tg_pad,
                to_pad, b_n1):
    return pl.pallas_call(
        _lut_kernel,
        out_shape=(
            jax.ShapeDtypeStruct((CPAD, H), jnp.float32),
            jax.ShapeDtypeStruct((8, H), jnp.float32),
            jax.ShapeDtypeStruct((16, H), jnp.float32),
            jax.ShapeDtypeStruct((16, H), jnp.float32),
            jax.ShapeDtypeStruct((16, H), jnp.float32),
        ),
    )(combo_pad, w_leaf, b_leaf.reshape(1, H), w_n1, tt_pad, te_pad, tg_pad,
      to_pad, b_n1.reshape(1, H))


def _onehot(idx, n):
    return (idx[:, None] == lax.broadcasted_iota(jnp.int32, (1, n), 1)
            ).astype(jnp.float32)


def _mlp_kernel(cs_ref, t_ref, a_ref, g_ref, o_ref, tw_ref, aw_ref, gw_ref,
                olut_ref, w2_ref, b2_ref, out_ref):
    dotd = lambda x, w: jnp.dot(x, w, preferred_element_type=jnp.float32)
    pre = (cs_ref[:, :H]
           + dotd(_onehot(t_ref[0, 0, :], 8), tw_ref[...])
           + dotd(_onehot(a_ref[0, 0, :], 16), aw_ref[...])
           + dotd(_onehot(g_ref[0, 0, :], 16), gw_ref[...])
           + dotd(_onehot(o_ref[0, 0, :], 16), olut_ref[...]))
    h = jnp.maximum(pre, 0.0)
    out_ref[...] = _dotf(h, w2_ref[...]) + b2_ref[...]


def _out_mlp(child_sum, trigger, action, target, operand_id, tw, aw, gw,
             olut, w2, b2):
    nb = NP // MBLK
    idx3 = lambda x: x.reshape(nb, 1, MBLK)
    ispec = pl.BlockSpec((1, 1, MBLK), lambda i: (i, 0, 0))
    return pl.pallas_call(
        _mlp_kernel,
        grid=(nb,),
        in_specs=[
            pl.BlockSpec((MBLK, AW), lambda i: (i, 0)),
            ispec, ispec, ispec, ispec,
            pl.BlockSpec((8, H), lambda i: (0, 0)),
            pl.BlockSpec((16, H), lambda i: (0, 0)),
            pl.BlockSpec((16, H), lambda i: (0, 0)),
            pl.BlockSpec((16, H), lambda i: (0, 0)),
            pl.BlockSpec((H, H), lambda i: (0, 0)),
            pl.BlockSpec((1, H), lambda i: (0, 0)),
        ],
        out_specs=pl.BlockSpec((MBLK, H), lambda i: (i, 0)),
        out_shape=jax.ShapeDtypeStruct((NP, H), jnp.float32),
    )(child_sum, idx3(trigger), idx3(action), idx3(target), idx3(operand_id),
      tw, aw, gw, olut, w2, b2.reshape(1, H))


def _iota16():
    return lax.broadcasted_iota(jnp.int32, (LANES,), 0)


def _sc_child_body(flut_hbm, ct_hbm, ca_hbm, cg_hbm, cp_hbm, starts_hbm,
                   out_hbm, flut_v, ct_v, ca_v, cg_v, cp_v, ct2_v, ca2_v,
                   cg2_v, cp2_v, st_v, acc, sem, sem2):
    wid = lax.axis_index("s") * 2 + lax.axis_index("c")
    pltpu.sync_copy(flut_hbm, flut_v)
    pltpu.sync_copy(starts_hbm, st_v)

    def issue(cbase, bt, ba_, bg, bp, sm):
        pltpu.async_copy(ct_hbm.at[pl.ds(cbase, CH)], bt, sm)
        pltpu.async_copy(ca_hbm.at[pl.ds(cbase, CH)], ba_, sm)
        pltpu.async_copy(cg_hbm.at[pl.ds(cbase, CH)], bg, sm)
        return pltpu.async_copy(cp_hbm.at[pl.ds(cbase, CH)], bp, sm)

    def outer(kb, carry_o):
        b = wid * (NBLK // NW) + kb
        p0 = pl.multiple_of(b * PB, PB)

        sv = st_v[pl.ds(b, LANES)]
        s = sv[0]
        e = sv[1]
        c0 = s & jnp.int32(~(LANES - 1))
        nch = (e - c0 + (CH - 1)) // CH
        d0 = issue(pl.multiple_of(c0, LANES), ct_v, ca_v, cg_v, cp_v, sem)

        def zloop(z, carry_z):
            base = z * AW
            for u in range(6):
                acc[pl.ds(base + u * LANES, LANES)] = jnp.zeros(
                    (LANES,), jnp.float32)
            return carry_z

        lax.fori_loop(0, PB, zloop, 0)
        zreg = jnp.zeros((LANES,), jnp.float32)

        def process(cbase, bt, ba_, bg, bp, carry):
            def grp(i, carry2):
                prev = carry2[0]
                regs = list(carry2[1:])
                pl16 = bp[pl.ds(i * LANES, LANES)] - p0
                bad = (pl16 < 0) | (pl16 >= PB)
                pb16 = jnp.where(bad, PB, pl16) * AW
                cb16 = (bt[pl.ds(i * LANES, LANES)] * 90
                        + ba_[pl.ds(i * LANES, LANES)] * 10
                        + bg[pl.ds(i * LANES, LANES)]) * H
                for k2 in range(LANES):
                    ck = cb16[k2]
                    pk = pb16[k2]
                    fresh = pk != prev
                    vj = [flut_v[pl.ds(ck + 16 * j, LANES)]
                          for j in range(6)]
                    regs = [jnp.where(fresh, vj[j], regs[j] + vj[j])
                            for j in range(6)]
                    for j in range(6):
                        acc[pl.ds(pk + 16 * j, LANES)] = regs[j]
                    prev = pk
                return (prev, regs[0], regs[1], regs[2], regs[3], regs[4],
                        regs[5])

            ngrp = jnp.minimum((e - cbase + (LANES - 1)) // LANES,
                               CH // LANES)
            return lax.fori_loop(0, jnp.maximum(ngrp, 0), grp, carry)

        def pair(k2, carry):
            ka = 2 * k2
            cba = pl.multiple_of(c0 + ka * CH, LANES)
            d0.wait()
            d0.wait()
            d0.wait()
            d0.wait()
            db = issue(pl.multiple_of(c0 + (ka + 1) * CH, LANES),
                       ct2_v, ca2_v, cg2_v, cp2_v, sem2)
            carry = process(cba, ct_v, ca_v, cg_v, cp_v, carry)
            db.wait()
            db.wait()
            db.wait()
            db.wait()
            issue(pl.multiple_of(c0 + (ka + 2) * CH, LANES),
                  ct_v, ca_v, cg_v, cp_v, sem)
            carry = process(cba + CH, ct2_v, ca2_v, cg2_v, cp2_v, carry)
            return carry

        carry0 = (jnp.int32(PB * AW), zreg, zreg, zreg, zreg, zreg, zreg)
        lax.fori_loop(0, (nch + 1) // 2, pair, carry0)
        d0.wait()
        d0.wait()
        d0.wait()
        d0.wait()
        pltpu.sync_copy(acc.at[pl.ds(0, PB * AW)],
                        out_hbm.at[pl.ds(p0 * AW, PB * AW)])
        return carry_o

    lax.fori_loop(0, NBLK // NW, outer, 0)


def _sc_child(flut_pad, ct, ca, cg, cp, starts):
    mesh = plsc.VectorSubcoreMesh(core_axis_name="c", subcore_axis_name="s",
                                  num_cores=2, num_subcores=16)
    fn = pl.kernel(
        _sc_child_body,
        out_type=jax.ShapeDtypeStruct((NP * AW,), jnp.float32),
        mesh=mesh,
        scratch_types=[
            pltpu.VMEM((CPAD * H,), jnp.float32),
            pltpu.VMEM((CH,), jnp.int32),
            pltpu.VMEM((CH,), jnp.int32),
            pltpu.VMEM((CH,), jnp.int32),
            pltpu.VMEM((CH,), jnp.int32),
            pltpu.VMEM((CH,), jnp.int32),
            pltpu.VMEM((CH,), jnp.int32),
            pltpu.VMEM((CH,), jnp.int32),
            pltpu.VMEM((CH,), jnp.int32),
            pltpu.VMEM((NBLK + LANES,), jnp.int32),
            pltpu.VMEM(((PB + 1) * AW,), jnp.float32),
            pltpu.SemaphoreType.DMA,
            pltpu.SemaphoreType.DMA,
        ],
        compiler_params=pltpu.CompilerParams(needs_layout_passes=False),
    )
    return fn(flut_pad, ct, ca, cg, cp, starts)


def kernel(trigger, action, target, operand_id, child_trigger, child_action,
           child_target, child_parent, T_trigger, T_effect, T_target,
           T_operand, W_leaf, b_leaf, W_n1, b_n1, W_n2, b_n2):
    tt = T_trigger.at[0].set(0.0)
    te = T_effect.at[0].set(0.0)
    tg = T_target.at[0].set(0.0)
    to = T_operand.at[0].set(0.0)
    combo = (tt[:, None, None, :] + te[None, :, None, :]
             + tg[None, None, :, :]).reshape(NCOMBO, H)
    combo_pad = jnp.zeros((CPAD, H), jnp.float32).at[:NCOMBO].set(combo)
    to_pad = jnp.zeros((16, H), jnp.float32).at[:15].set(to)
    tt_pad = jnp.zeros((8, H), jnp.float32).at[:7].set(tt)
    te_pad = jnp.zeros((16, H), jnp.float32).at[:9].set(te)
    tg_pad = jnp.zeros((16, H), jnp.float32).at[:10].set(tg)

    flut, tw, aw, gw, olut = _build_luts(combo_pad, W_leaf, b_leaf, W_n1,
                                         tt_pad, te_pad, tg_pad, to_pad, b_n1)

    bounds = jnp.arange(0, NP + 1, PB, dtype=jnp.int32)
    starts = jnp.searchsorted(child_parent, bounds, side="left",
                              method="compare_all").astype(jnp.int32)
    starts_pad = jnp.zeros((NBLK + LANES,), jnp.int32).at[:NBLK + 1].set(
        starts)
    ct_pad = jnp.concatenate([child_trigger, jnp.zeros((4 * CH,), jnp.int32)])
    ca_pad = jnp.concatenate([child_action, jnp.zeros((4 * CH,), jnp.int32)])
    cg_pad = jnp.concatenate([child_target, jnp.zeros((4 * CH,), jnp.int32)])
    cp_pad = jnp.concatenate([child_parent,
                              jnp.full((4 * CH,), NP, jnp.int32)])

    child_sum = _sc_child(flut.reshape(-1), ct_pad, ca_pad, cg_pad,
                          cp_pad, starts_pad)

    return _out_mlp(child_sum.reshape(NP, AW), trigger, action, target,
                    operand_id, tw, aw, gw, olut, W_n2, b_n2)


# speedup vs baseline: 16.1875x; 1.0006x over previous
import jax
import jax.numpy as jnp
from jax import lax
from jax.experimental import pallas as pl
from jax.experimental.pallas import tpu as pltpu
from jax.experimental.pallas import tpu_sc as plsc

H = 96
AW = 128
NP = 65536
NC = 262144
NCOMBO = 630
CPAD = 640
NW = 32
PB = 512
NBLK = NP // PB
CH = 384
LANES = 16
MBLK = 8192


def _dotf(x, w):
    return jnp.dot(x, w, preferred_element_type=jnp.float32,
                   precision=lax.Precision.HIGHEST)


def _lut_kernel(combo_ref, wl_ref, bl_ref, wn1_ref, tt_ref, te_ref, tg_ref,
                to_ref, bn1_ref, flut_ref, tw_ref, aw_ref, gw_ref, olut_ref):
    combo = combo_ref[...]
    leaf = jnp.maximum(_dotf(combo, wl_ref[...]) + bl_ref[...], 0.0)
    wb = wn1_ref[H:, :]
    wa = wn1_ref[:H, :]
    flut_ref[...] = _dotf(leaf, wb)
    tw_ref[...] = _dotf(tt_ref[...], wb)
    aw_ref[...] = _dotf(te_ref[...], wb)
    gw_ref[...] = _dotf(tg_ref[...], wb)
    olut_ref[...] = _dotf(to_ref[...], wa) + bn1_ref[...]


def _build_luts(combo_pad, w_leaf, b_leaf, w_n1, tt_pad, te_pad, tg_pad,
                to_pad, b_n1):
    return pl.pallas_call(
        _lut_kernel,
        out_shape=(
            jax.ShapeDtypeStruct((CPAD, H), jnp.float32),
            jax.ShapeDtypeStruct((8, H), jnp.float32),
            jax.ShapeDtypeStruct((16, H), jnp.float32),
            jax.ShapeDtypeStruct((16, H), jnp.float32),
            jax.ShapeDtypeStruct((16, H), jnp.float32),
        ),
    )(combo_pad, w_leaf, b_leaf.reshape(1, H), w_n1, tt_pad, te_pad, tg_pad,
      to_pad, b_n1.reshape(1, H))


def _onehot(idx, n):
    return (idx[:, None] == lax.broadcasted_iota(jnp.int32, (1, n), 1)
            ).astype(jnp.float32)


def _mlp_kernel(cs_ref, t_ref, a_ref, g_ref, o_ref, tw_ref, aw_ref, gw_ref,
                olut_ref, w2_ref, b2_ref, out_ref):
    dotd = lambda x, w: jnp.dot(x, w, preferred_element_type=jnp.float32)
    cs = cs_ref[...].reshape(MBLK, AW)
    pre = (cs[:, :H]
           + dotd(_onehot(t_ref[0, 0, :], 8), tw_ref[...])
           + dotd(_onehot(a_ref[0, 0, :], 16), aw_ref[...])
           + dotd(_onehot(g_ref[0, 0, :], 16), gw_ref[...])
           + dotd(_onehot(o_ref[0, 0, :], 16), olut_ref[...]))
    h = jnp.maximum(pre, 0.0)
    out_ref[...] = _dotf(h, w2_ref[...]) + b2_ref[...]


def _out_mlp(child_sum, trigger, action, target, operand_id, tw, aw, gw,
             olut, w2, b2):
    nb = NP // MBLK
    idx3 = lambda x: x.reshape(nb, 1, MBLK)
    ispec = pl.BlockSpec((1, 1, MBLK), lambda i: (i, 0, 0))
    return pl.pallas_call(
        _mlp_kernel,
        grid=(nb,),
        in_specs=[
            pl.BlockSpec((MBLK * AW,), lambda i: (i,)),
            ispec, ispec, ispec, ispec,
            pl.BlockSpec((8, H), lambda i: (0, 0)),
            pl.BlockSpec((16, H), lambda i: (0, 0)),
            pl.BlockSpec((16, H), lambda i: (0, 0)),
            pl.BlockSpec((16, H), lambda i: (0, 0)),
            pl.BlockSpec((H, H), lambda i: (0, 0)),
            pl.BlockSpec((1, H), lambda i: (0, 0)),
        ],
        out_specs=pl.BlockSpec((MBLK, H), lambda i: (i, 0)),
        out_shape=jax.ShapeDtypeStruct((NP, H), jnp.float32),
    )(child_sum.reshape(-1), idx3(trigger), idx3(action), idx3(target),
      idx3(operand_id),
      tw, aw, gw, olut, w2, b2.reshape(1, H))


def _iota16():
    return lax.broadcasted_iota(jnp.int32, (LANES,), 0)


def _sc_child_body(flut_hbm, ct_hbm, ca_hbm, cg_hbm, cp_hbm, starts_hbm,
                   out_hbm, flut_v, ct_v, ca_v, cg_v, cp_v, ct2_v, ca2_v,
                   cg2_v, cp2_v, st_v, acc, sem, sem2):
    wid = lax.axis_index("s") * 2 + lax.axis_index("c")
    pltpu.sync_copy(flut_hbm, flut_v)
    pltpu.sync_copy(starts_hbm, st_v)

    def issue(cbase, bt, ba_, bg, bp, sm):
        pltpu.async_copy(ct_hbm.at[pl.ds(cbase, CH)], bt, sm)
        pltpu.async_copy(ca_hbm.at[pl.ds(cbase, CH)], ba_, sm)
        pltpu.async_copy(cg_hbm.at[pl.ds(cbase, CH)], bg, sm)
        return pltpu.async_copy(cp_hbm.at[pl.ds(cbase, CH)], bp, sm)

    def outer(kb, carry_o):
        b = wid * (NBLK // NW) + kb
        p0 = pl.multiple_of(b * PB, PB)

        sv = st_v[pl.ds(b, LANES)]
        s = sv[0]
        e = sv[1]
        c0 = s & jnp.int32(~(LANES - 1))
        nch = (e - c0 + (CH - 1)) // CH
        d0 = issue(pl.multiple_of(c0, LANES), ct_v, ca_v, cg_v, cp_v, sem)

        def zloop(z, carry_z):
            base = z * AW
            for u in range(6):
                acc[pl.ds(base + u * LANES, LANES)] = jnp.zeros(
                    (LANES,), jnp.float32)
            return carry_z

        lax.fori_loop(0, PB, zloop, 0)
        zreg = jnp.zeros((LANES,), jnp.float32)

        def process(cbase, bt, ba_, bg, bp, carry):
            def grp(i, carry2):
                prev = carry2[0]
                regs = list(carry2[1:])
                pl16 = bp[pl.ds(i * LANES, LANES)] - p0
                bad = (pl16 < 0) | (pl16 >= PB)
                pb16 = jnp.where(bad, PB, pl16) * AW
                cb16 = (bt[pl.ds(i * LANES, LANES)] * 90
                        + ba_[pl.ds(i * LANES, LANES)] * 10
                        + bg[pl.ds(i * LANES, LANES)]) * H
                for k2 in range(LANES):
                    ck = cb16[k2]
                    pk = pb16[k2]
                    fresh = pk != prev
                    vj = [flut_v[pl.ds(ck + 16 * j, LANES)]
                          for j in range(6)]
                    regs = [jnp.where(fresh, vj[j], regs[j] + vj[j])
                            for j in range(6)]
                    for j in range(6):
                        acc[pl.ds(pk + 16 * j, LANES)] = regs[j]
                    prev = pk
                return (prev, regs[0], regs[1], regs[2], regs[3], regs[4],
                        regs[5])

            ngrp = jnp.minimum((e - cbase + (LANES - 1)) // LANES,
                               CH // LANES)
            return lax.fori_loop(0, jnp.maximum(ngrp, 0), grp, carry)

        def pair(k2, carry):
            ka = 2 * k2
            cba = pl.multiple_of(c0 + ka * CH, LANES)
            d0.wait()
            d0.wait()
            d0.wait()
            d0.wait()
            db = issue(pl.multiple_of(c0 + (ka + 1) * CH, LANES),
                       ct2_v, ca2_v, cg2_v, cp2_v, sem2)
            carry = process(cba, ct_v, ca_v, cg_v, cp_v, carry)
            db.wait()
            db.wait()
            db.wait()
            db.wait()
            issue(pl.multiple_of(c0 + (ka + 2) * CH, LANES),
                  ct_v, ca_v, cg_v, cp_v, sem)
            carry = process(cba + CH, ct2_v, ca2_v, cg2_v, cp2_v, carry)
            return carry

        carry0 = (jnp.int32(PB * AW), zreg, zreg, zreg, zreg, zreg, zreg)
        lax.fori_loop(0, (nch + 1) // 2, pair, carry0)
        d0.wait()
        d0.wait()
        d0.wait()
        d0.wait()
        pltpu.sync_copy(acc.at[pl.ds(0, PB * AW)],
                        out_hbm.at[pl.ds(p0 * AW, PB * AW)])
        return carry_o

    lax.fori_loop(0, NBLK // NW, outer, 0)


def _sc_child(flut_pad, ct, ca, cg, cp, starts):
    mesh = plsc.VectorSubcoreMesh(core_axis_name="c", subcore_axis_name="s",
                                  num_cores=2, num_subcores=16)
    fn = pl.kernel(
        _sc_child_body,
        out_type=jax.ShapeDtypeStruct((NP * AW,), jnp.float32),
        mesh=mesh,
        scratch_types=[
            pltpu.VMEM((CPAD * H,), jnp.float32),
            pltpu.VMEM((CH,), jnp.int32),
            pltpu.VMEM((CH,), jnp.int32),
            pltpu.VMEM((CH,), jnp.int32),
            pltpu.VMEM((CH,), jnp.int32),
            pltpu.VMEM((CH,), jnp.int32),
            pltpu.VMEM((CH,), jnp.int32),
            pltpu.VMEM((CH,), jnp.int32),
            pltpu.VMEM((CH,), jnp.int32),
            pltpu.VMEM((NBLK + LANES,), jnp.int32),
            pltpu.VMEM(((PB + 1) * AW,), jnp.float32),
            pltpu.SemaphoreType.DMA,
            pltpu.SemaphoreType.DMA,
        ],
        compiler_params=pltpu.CompilerParams(needs_layout_passes=False),
    )
    return fn(flut_pad, ct, ca, cg, cp, starts)


def kernel(trigger, action, target, operand_id, child_trigger, child_action,
           child_target, child_parent, T_trigger, T_effect, T_target,
           T_operand, W_leaf, b_leaf, W_n1, b_n1, W_n2, b_n2):
    tt = T_trigger.at[0].set(0.0)
    te = T_effect.at[0].set(0.0)
    tg = T_target.at[0].set(0.0)
    to = T_operand.at[0].set(0.0)
    combo = (tt[:, None, None, :] + te[None, :, None, :]
             + tg[None, None, :, :]).reshape(NCOMBO, H)
    combo_pad = jnp.zeros((CPAD, H), jnp.float32).at[:NCOMBO].set(combo)
    to_pad = jnp.zeros((16, H), jnp.float32).at[:15].set(to)
    tt_pad = jnp.zeros((8, H), jnp.float32).at[:7].set(tt)
    te_pad = jnp.zeros((16, H), jnp.float32).at[:9].set(te)
    tg_pad = jnp.zeros((16, H), jnp.float32).at[:10].set(tg)

    flut, tw, aw, gw, olut = _build_luts(combo_pad, W_leaf, b_leaf, W_n1,
                                         tt_pad, te_pad, tg_pad, to_pad, b_n1)

    bounds = jnp.arange(0, NP + 1, PB, dtype=jnp.int32)
    starts = jnp.searchsorted(child_parent, bounds, side="left",
                              method="compare_all").astype(jnp.int32)
    starts_pad = jnp.zeros((NBLK + LANES,), jnp.int32).at[:NBLK + 1].set(
        starts)
    ct_pad = jnp.concatenate([child_trigger, jnp.zeros((4 * CH,), jnp.int32)])
    ca_pad = jnp.concatenate([child_action, jnp.zeros((4 * CH,), jnp.int32)])
    cg_pad = jnp.concatenate([child_target, jnp.zeros((4 * CH,), jnp.int32)])
    cp_pad = jnp.concatenate([child_parent,
                              jnp.full((4 * CH,), NP, jnp.int32)])

    child_sum = _sc_child(flut.reshape(-1), ct_pad, ca_pad, cg_pad,
                          cp_pad, starts_pad)

    return _out_mlp(child_sum, trigger, action, target,
                    operand_id, tw, aw, gw, olut, W_n2, b_n2)
